# baseline (speedup 1.0000x reference)
import jax
import jax.numpy as jnp
from jax import lax
from jax.experimental import pallas as pl
from jax.experimental.pallas import tpu as pltpu
from jax.experimental.pallas import tpu_sc as plsc

_B, _S, _D = 4, 2048, 768
_TC_B = 3
_NC, _NS = 2, 16
_NW = _NC * _NS
_ROWS_PER_W = _S // _NW
_K = 32
_NV = _D // 16


def _tc_add(x_ref, p_ref, o_ref):
    o_ref[...] = x_ref[...] + p_ref[...][None, :, :]


def _tc_part(inputs, pos_table):
    return pl.pallas_call(
        _tc_add,
        grid=(_TC_B,),
        in_specs=[
            pl.BlockSpec((1, _S, _D), lambda b: (b, 0, 0)),
            pl.BlockSpec((_S, _D), lambda b: (0, 0)),
        ],
        out_specs=pl.BlockSpec((1, _S, _D), lambda b: (b, 0, 0)),
        out_shape=jax.ShapeDtypeStruct((_TC_B, _S, _D), inputs.dtype),
        compiler_params=pltpu.CompilerParams(
            dimension_semantics=("parallel",),
        ),
    )(inputs, pos_table)


def _sc_body(x_hbm, pos_hbm, out_hbm, p_buf0, p_buf1, x_buf0, x_buf1,
             sem_x0, sem_x1):
    w = lax.axis_index("s") * _NC + lax.axis_index("c")
    base = w * _ROWS_PER_W

    p_bufs = (p_buf0, p_buf1)
    x_bufs = (x_buf0, x_buf1)
    sems_x = (sem_x0, sem_x1)

    pltpu.sync_copy(pos_hbm.at[pl.ds(base, _K)], p_buf0)
    pltpu.sync_copy(pos_hbm.at[pl.ds(base + _K, _K)], p_buf1)

    seq = [(c, b) for c in range(2) for b in range(_TC_B, _B)]

    def start_load(g):
        c, b = seq[g]
        return pltpu.async_copy(
            x_hbm.at[b, pl.ds(base + c * _K, _K)], x_bufs[g % 2], sems_x[g % 2]
        )

    pending = start_load(0)
    for g, (c, b) in enumerate(seq):
        cur_copy = pending
        if g + 1 < len(seq):
            pending = start_load(g + 1)
        cur_copy.wait()

        p_buf = p_bufs[c]
        x_buf = x_bufs[g % 2]

        def add_row(r, _, x_buf=x_buf, p_buf=p_buf):
            for j in range(_NV):
                x_buf[r, pl.ds(j * 16, 16)] = (
                    x_buf[r, pl.ds(j * 16, 16)] + p_buf[r, pl.ds(j * 16, 16)]
                )
            return 0

        lax.fori_loop(0, _K, add_row, 0)
        pltpu.sync_copy(x_buf, out_hbm.at[b - _TC_B, pl.ds(base + c * _K, _K)])


def _sc_part(inputs, pos_table):
    return pl.kernel(
        _sc_body,
        out_type=jax.ShapeDtypeStruct((_B - _TC_B, _S, _D), jnp.float32),
        mesh=plsc.VectorSubcoreMesh(core_axis_name="c", subcore_axis_name="s"),
        scratch_types=[
            pltpu.VMEM((_K, _D), jnp.float32),
            pltpu.VMEM((_K, _D), jnp.float32),
            pltpu.VMEM((_K, _D), jnp.float32),
            pltpu.VMEM((_K, _D), jnp.float32),
            pltpu.SemaphoreType.DMA,
            pltpu.SemaphoreType.DMA,
        ],
    )(inputs, pos_table)


def kernel(inputs, pos_table):
    tc_out = _tc_part(inputs, pos_table)
    sc_out = _sc_part(inputs, pos_table)
    return jnp.concatenate([tc_out, sc_out], axis=0)


# speedup vs baseline: 3.0734x; 3.0734x over previous
import jax
import jax.numpy as jnp
from jax.experimental import pallas as pl
from jax.experimental.pallas import tpu as pltpu


def _add_kernel(x_ref, p_ref, o_ref):
    o_ref[...] = x_ref[...] + p_ref[...][None, :, :]


def kernel(inputs, pos_table):
    B, S, D = inputs.shape
    BS = 1024
    return pl.pallas_call(
        _add_kernel,
        grid=(S // BS,),
        in_specs=[
            pl.BlockSpec((B, BS, D), lambda i: (0, i, 0)),
            pl.BlockSpec((BS, D), lambda i: (i, 0)),
        ],
        out_specs=pl.BlockSpec((B, BS, D), lambda i: (0, i, 0)),
        out_shape=jax.ShapeDtypeStruct((B, S, D), inputs.dtype),
        compiler_params=pltpu.CompilerParams(
            dimension_semantics=("parallel",),
        ),
    )(inputs, pos_table)
